# baseline (speedup 1.0000x reference)
import functools

import jax
import jax.numpy as jnp
from jax import lax
from jax.experimental import pallas as pl
from jax.experimental.pallas import tpu as pltpu
from jax.experimental.pallas import tpu_sc as plsc

_B, _L, _H = 1, 2048, 768
_E, _TOP_K, _FF = 16, 2, 512
_KCONV = 3
_EPS = 1e-05

_BLK1 = 256
_BLK2 = 256
_MAXS = _L * _TOP_K + _E * _BLK2
_NBLK = _MAXS // _BLK2
_NW = 32
_TPW = _L // _NW
_SPW = _MAXS // _NW

_mesh = plsc.VectorSubcoreMesh(core_axis_name="c", subcore_axis_name="s")
_sc_params = pltpu.CompilerParams(needs_layout_passes=False)


def _rms(x, w):
    return x * jax.lax.rsqrt(jnp.mean(x * x, axis=-1, keepdims=True) + _EPS) * w


def _stage1_body(x_ref, opw_ref, ffnw_ref, win_ref, wconvt_ref, wout_ref,
                 wr_ref, bias_ref, wg_ref, wu_ref, wd_ref,
                 h_ref, t_ref, comb_ref, wgb_ref, wub_ref, wdb_ref,
                 carry_ref, winb_ref, woutb_ref):
    i = pl.program_id(0)

    @pl.when(i == 0)
    def _():
        carry_ref[...] = jnp.zeros_like(carry_ref)
        winb_ref[...] = win_ref[...].astype(jnp.bfloat16)
        woutb_ref[...] = wout_ref[...].astype(jnp.bfloat16)

    wgb_ref[...] = wg_ref[...].astype(jnp.bfloat16)
    wub_ref[...] = wu_ref[...].astype(jnp.bfloat16)
    wdb_ref[...] = wd_ref[...].astype(jnp.bfloat16)

    xb = x_ref[...]
    hn = _rms(xb, opw_ref[...])
    bcx = jnp.dot(hn.astype(jnp.bfloat16), winb_ref[...],
                  preferred_element_type=jnp.float32)
    b_ = bcx[:, :_H]
    c_ = bcx[:, _H:2 * _H]
    xc = bcx[:, 2 * _H:]
    Bx = b_ * xc

    carry = carry_ref[...]
    sh1 = jnp.concatenate([carry[7:8], Bx[:-1]], axis=0)
    sh2 = jnp.concatenate([carry[6:8], Bx[:-2]], axis=0)[:_BLK1]
    w0 = wconvt_ref[0:1, :]
    w1 = wconvt_ref[1:2, :]
    w2 = wconvt_ref[2:3, :]
    conv = sh2 * w0 + sh1 * w1 + Bx * w2
    carry_ref[...] = Bx[-8:]

    y = c_ * conv
    r = jnp.dot(y.astype(jnp.bfloat16), woutb_ref[...],
                preferred_element_type=jnp.float32)
    h = xb + r
    h_ref[...] = h

    t = _rms(h, ffnw_ref[...])
    t_ref[...] = t

    logits = jnp.dot(t, wr_ref[...], preferred_element_type=jnp.float32)
    m = jnp.max(logits, axis=-1, keepdims=True)
    p = jnp.exp(logits - m)
    g = p / jnp.sum(p, axis=-1, keepdims=True) + bias_ref[...]

    iota = jax.lax.broadcasted_iota(jnp.int32, g.shape, 1)
    m0 = jnp.max(g, axis=-1, keepdims=True)
    i0 = jnp.min(jnp.where(g == m0, iota, _E), axis=-1, keepdims=True)
    oh0 = iota == i0
    g1 = jnp.where(oh0, -jnp.inf, g)
    m1 = jnp.max(g1, axis=-1, keepdims=True)
    i1 = jnp.min(jnp.where(g1 == m1, iota, _E), axis=-1, keepdims=True)
    oh1 = iota == i1
    denom = m0 + m1 + 1e-20
    comb_ref[...] = (jnp.where(oh0, m0, 0.0) + jnp.where(oh1, m1, 0.0)) / denom


def _wid():
    return lax.axis_index("s") * 2 + lax.axis_index("c")


def _iota16():
    return lax.iota(jnp.int32, 16)


def _route_body(comb_hbm, svals_hbm, eids_hbm, cnts_hbm,
                comb_v, svals_v, eids_v, cnt_v, sem):
    w = _wid()
    tok0 = w * _TPW
    pltpu.sync_copy(comb_hbm.at[pl.ds(tok0, _TPW)], comb_v)
    iota = _iota16()
    mask2 = iota < _TOP_K
    cnt_v[...] = jnp.zeros((16,), jnp.int32)
    ones = jnp.ones((16,), jnp.int32)
    for j in range(_TPW):
        row = comb_v[j]
        sv, se = plsc.sort_key_val(row, iota, descending=True)
        tot = jnp.sum(jnp.where(mask2, sv, 0.0))
        svals_v[j] = sv / (tot + 1e-20)
        eids_v[j] = se
        plsc.addupdate_scatter(cnt_v, [se], ones, mask=mask2)
    pltpu.sync_copy(svals_v, svals_hbm.at[pl.ds(tok0, _TPW)])
    pltpu.sync_copy(eids_v, eids_hbm.at[pl.ds(tok0, _TPW)])
    pltpu.sync_copy(cnt_v, cnts_hbm.at[w])


def _slot_body(cnts_hbm, eids_hbm, t_hbm, slotmap_hbm, blkinfo_hbm, td_hbm,
               cnts_v, eids_v, cntrun_v, slotmap_v, s0pad_v, s1pad_v,
               slot0_v, slot1_v, trows_v, blk_v, nb_v, sem, sem2):
    w = _wid()
    tok0 = w * _TPW
    cp_t = pltpu.async_copy(t_hbm.at[pl.ds(tok0, _TPW)], trows_v, sem)
    pltpu.sync_copy(cnts_hbm, cnts_v)
    pltpu.sync_copy(eids_hbm.at[pl.ds(tok0, _TPW)], eids_v)
    iota = _iota16()
    mask2 = iota < _TOP_K
    ones = jnp.ones((16,), jnp.int32)
    zeros16 = jnp.zeros((16,), jnp.int32)

    gcnt = jnp.zeros((16,), jnp.int32)
    pref = jnp.zeros((16,), jnp.int32)
    for wp in range(_NW):
        rowc = cnts_v[wp]
        gcnt = gcnt + rowc
        pref = pref + jnp.where(wp < w, rowc, zeros16)
    pc = ((gcnt + (_BLK2 - 1)) >> 8) << 8
    base = plsc.cumsum(pc) - pc
    mybase = base + pref
    cntrun_v[...] = jnp.zeros((16,), jnp.int32)

    for j in range(_TPW):
        ev = eids_v[j]
        r = plsc.load_gather(cntrun_v, [ev], mask=mask2)
        b = jnp.take_along_axis(mybase, ev, axis=0)
        plsc.addupdate_scatter(cntrun_v, [ev], ones, mask=mask2)
        slotv = b + r
        slotmap_v[j] = slotv
        plsc.store_compressed(s0pad_v.at[pl.ds(j, 16)], slotv, mask=iota == 0)
        plsc.store_compressed(s1pad_v.at[pl.ds(j, 16)], slotv, mask=iota == 1)

    for g in range(_TPW // 16):
        slot0_v[pl.ds(16 * g, 16)] = s0pad_v[pl.ds(16 * g, 16)]
        slot1_v[pl.ds(16 * g, 16)] = s1pad_v[pl.ds(16 * g, 16)]

    pltpu.sync_copy(slotmap_v, slotmap_hbm.at[pl.ds(tok0, _TPW)])
    cp_t.wait()
    c0 = pltpu.async_copy(trows_v, td_hbm.at[slot0_v], sem)
    c1 = pltpu.async_copy(trows_v, td_hbm.at[slot1_v], sem2)

    @pl.when(w == 0)
    def _():
        bstart = base >> 8
        for c in range(_NBLK // 16):
            bvec = iota + 16 * c
            cntv = jnp.zeros((16,), jnp.int32)
            for e in range(_E):
                bse = jnp.take_along_axis(
                    bstart, jnp.full((16,), e, jnp.int32), axis=0)
                cntv = cntv + jnp.where(bvec >= bse, 1, 0).astype(jnp.int32)
            blk_v[pl.ds(16 * c, 16)] = jnp.clip(cntv - 1, 0, _E - 1)
        nb = jnp.sum(pc) >> 8
        nb_v[...] = jnp.full((16,), nb, jnp.int32)
        pltpu.sync_copy(blk_v, blkinfo_hbm.at[pl.ds(0, _NBLK)])
        pltpu.sync_copy(nb_v, blkinfo_hbm.at[pl.ds(64, 16)])

    c0.wait()
    c1.wait()


def _ffn_body(binfo_ref, td_ref, wg_ref, wu_ref, wd_ref, y_ref):
    i = pl.program_id(0)

    @pl.when(i < binfo_ref[64])
    def _():
        td = td_ref[...].astype(jnp.bfloat16)
        a = jnp.dot(td, wg_ref[0], preferred_element_type=jnp.float32)
        u = jnp.dot(td, wu_ref[0], preferred_element_type=jnp.float32)
        he = (a / (1.0 + jnp.exp(-a))) * u
        y_ref[...] = jnp.dot(he.astype(jnp.bfloat16), wd_ref[0],
                             preferred_element_type=jnp.float32)


def _combine_body(slotmap_hbm, svals_hbm, y_hbm, h_hbm, out_hbm,
                  slm_v, sv_v, sl2_v, yr_v, h_v, out_v,
                  ssv, sy0, sy1, sh0, sh1, so0, so1):
    w = _wid()
    iota = _iota16()
    tokb = w * _TPW
    cps = pltpu.async_copy(svals_hbm.at[pl.ds(tokb, _TPW)], sv_v, ssv)
    pltpu.sync_copy(slotmap_hbm.at[pl.ds(tokb, _TPW)], slm_v)
    for c in range(4):
        for gg in range(2):
            g = 2 * c + gg
            rvec = (iota >> 1) + 8 * g
            sl2_v[c, pl.ds(16 * gg, 16)] = plsc.load_gather(
                slm_v, [rvec, iota & 1])

    sy = (sy0, sy1)
    sh = (sh0, sh1)
    so = (so0, so1)
    cpy = [None, None]
    cph = [None, None]
    cpo = [None, None]
    cpy[0] = pltpu.async_copy(y_hbm.at[sl2_v.at[0]], yr_v.at[0], sy[0])
    cph[0] = pltpu.async_copy(h_hbm.at[pl.ds(tokb, 16)], h_v.at[0], sh[0])
    cps.wait()
    for c in range(4):
        b = c & 1
        if c + 1 < 4:
            nb = (c + 1) & 1
            cpy[nb] = pltpu.async_copy(
                y_hbm.at[sl2_v.at[c + 1]], yr_v.at[nb], sy[nb])
            cph[nb] = pltpu.async_copy(
                h_hbm.at[pl.ds(tokb + 16 * (c + 1), 16)], h_v.at[nb], sh[nb])
        cpy[b].wait()
        cph[b].wait()
        if c >= 2:
            cpo[b].wait()

        def tok(j, carry, c=c, b=b):
            s0 = plsc.load_gather(sv_v, [jnp.full((16,), c * 16 + j, jnp.int32),
                                         jnp.zeros((16,), jnp.int32)])
            s1 = plsc.load_gather(sv_v, [jnp.full((16,), c * 16 + j, jnp.int32),
                                         jnp.ones((16,), jnp.int32)])
            for k in range(_H // 16):
                sl = pl.ds(16 * k, 16)
                out_v[b, j, sl] = (h_v[b, j, sl] + s0 * yr_v[b, 2 * j, sl]
                                   + s1 * yr_v[b, 2 * j + 1, sl])
            return carry

        lax.fori_loop(0, 16, tok, 0)
        cpo[b] = pltpu.async_copy(
            out_v.at[b], out_hbm.at[pl.ds(tokb + 16 * c, 16)], so[b])
    cpo[0].wait()
    cpo[1].wait()


def kernel(x, op_norm_w, ffn_norm_w, w_in, w_conv, w_out, w_router,
           expert_bias, w_g, w_u, w_d):
    x2 = x.reshape(_L, _H)
    wconvt = jnp.zeros((8, _H), jnp.float32).at[:_KCONV].set(w_conv.T)

    nblk1 = _L // _BLK1
    h, t, comb, wgb, wub, wdb = pl.pallas_call(
        _stage1_body,
        grid=(nblk1,),
        in_specs=[
            pl.BlockSpec((_BLK1, _H), lambda i: (i, 0)),
            pl.BlockSpec((1, _H), lambda i: (0, 0)),
            pl.BlockSpec((1, _H), lambda i: (0, 0)),
            pl.BlockSpec((_H, 3 * _H), lambda i: (0, 0)),
            pl.BlockSpec((8, _H), lambda i: (0, 0)),
            pl.BlockSpec((_H, _H), lambda i: (0, 0)),
            pl.BlockSpec((_H, _E), lambda i: (0, 0)),
            pl.BlockSpec((1, _E), lambda i: (0, 0)),
            pl.BlockSpec((2, _H, _FF), lambda i: (i, 0, 0)),
            pl.BlockSpec((2, _H, _FF), lambda i: (i, 0, 0)),
            pl.BlockSpec((2, _FF, _H), lambda i: (i, 0, 0)),
        ],
        out_specs=[
            pl.BlockSpec((_BLK1, _H), lambda i: (i, 0)),
            pl.BlockSpec((_BLK1, _H), lambda i: (i, 0)),
            pl.BlockSpec((_BLK1, _E), lambda i: (i, 0)),
            pl.BlockSpec((2, _H, _FF), lambda i: (i, 0, 0)),
            pl.BlockSpec((2, _H, _FF), lambda i: (i, 0, 0)),
            pl.BlockSpec((2, _FF, _H), lambda i: (i, 0, 0)),
        ],
        out_shape=[
            jax.ShapeDtypeStruct((_L, _H), jnp.float32),
            jax.ShapeDtypeStruct((_L, _H), jnp.float32),
            jax.ShapeDtypeStruct((_L, _E), jnp.float32),
            jax.ShapeDtypeStruct((_E, _H, _FF), jnp.bfloat16),
            jax.ShapeDtypeStruct((_E, _H, _FF), jnp.bfloat16),
            jax.ShapeDtypeStruct((_E, _FF, _H), jnp.bfloat16),
        ],
        scratch_shapes=[
            pltpu.VMEM((8, _H), jnp.float32),
            pltpu.VMEM((_H, 3 * _H), jnp.bfloat16),
            pltpu.VMEM((_H, _H), jnp.bfloat16),
        ],
    )(
        x2,
        op_norm_w.reshape(1, _H),
        ffn_norm_w.reshape(1, _H),
        w_in,
        wconvt,
        w_out,
        w_router,
        expert_bias.reshape(1, _E),
        w_g,
        w_u,
        w_d,
    )

    svals, eids, cnts = pl.kernel(
        _route_body,
        mesh=_mesh,
        compiler_params=_sc_params,
        out_type=[
            jax.ShapeDtypeStruct((_L, 16), jnp.float32),
            jax.ShapeDtypeStruct((_L, 16), jnp.int32),
            jax.ShapeDtypeStruct((_NW, 16), jnp.int32),
        ],
        scratch_types=[
            pltpu.VMEM((_TPW, 16), jnp.float32),
            pltpu.VMEM((_TPW, 16), jnp.float32),
            pltpu.VMEM((_TPW, 16), jnp.int32),
            pltpu.VMEM((16,), jnp.int32),
            pltpu.SemaphoreType.DMA,
        ],
    )(comb)

    slotmap, blkinfo, td = pl.kernel(
        _slot_body,
        mesh=_mesh,
        compiler_params=_sc_params,
        out_type=[
            jax.ShapeDtypeStruct((_L, 16), jnp.int32),
            jax.ShapeDtypeStruct((128,), jnp.int32),
            jax.ShapeDtypeStruct((_MAXS, _H), jnp.float32),
        ],
        scratch_types=[
            pltpu.VMEM((_NW, 16), jnp.int32),
            pltpu.VMEM((_TPW, 16), jnp.int32),
            pltpu.VMEM((16,), jnp.int32),
            pltpu.VMEM((_TPW, 16), jnp.int32),
            pltpu.VMEM((_TPW + 16, ), jnp.int32),
            pltpu.VMEM((_TPW + 16, ), jnp.int32),
            pltpu.VMEM((_TPW,), jnp.int32),
            pltpu.VMEM((_TPW,), jnp.int32),
            pltpu.VMEM((_TPW, _H), jnp.float32),
            pltpu.VMEM((_NBLK,), jnp.int32),
            pltpu.VMEM((16,), jnp.int32),
            pltpu.SemaphoreType.DMA,
            pltpu.SemaphoreType.DMA,
        ],
    )(cnts, eids, t)

    y = pl.pallas_call(
        _ffn_body,
        grid_spec=pltpu.PrefetchScalarGridSpec(
            num_scalar_prefetch=1,
            grid=(_NBLK,),
            in_specs=[
                pl.BlockSpec((_BLK2, _H), lambda i, binfo: (i, 0)),
                pl.BlockSpec((1, _H, _FF), lambda i, binfo: (binfo[i], 0, 0)),
                pl.BlockSpec((1, _H, _FF), lambda i, binfo: (binfo[i], 0, 0)),
                pl.BlockSpec((1, _FF, _H), lambda i, binfo: (binfo[i], 0, 0)),
            ],
            out_specs=pl.BlockSpec((_BLK2, _H), lambda i, binfo: (i, 0)),
        ),
        out_shape=jax.ShapeDtypeStruct((_MAXS, _H), jnp.float32),
    )(blkinfo, td, wgb, wub, wdb)

    out = pl.kernel(
        _combine_body,
        mesh=_mesh,
        compiler_params=_sc_params,
        out_type=jax.ShapeDtypeStruct((_L, _H), jnp.float32),
        scratch_types=[
            pltpu.VMEM((_TPW, 16), jnp.int32),
            pltpu.VMEM((_TPW, 16), jnp.float32),
            pltpu.VMEM((4, 32), jnp.int32),
            pltpu.VMEM((2, 32, _H), jnp.float32),
            pltpu.VMEM((2, 16, _H), jnp.float32),
            pltpu.VMEM((2, 16, _H), jnp.float32),
            pltpu.SemaphoreType.DMA,
            pltpu.SemaphoreType.DMA,
            pltpu.SemaphoreType.DMA,
            pltpu.SemaphoreType.DMA,
            pltpu.SemaphoreType.DMA,
            pltpu.SemaphoreType.DMA,
            pltpu.SemaphoreType.DMA,
        ],
    )(slotmap, svals, y, h)

    return out.reshape(_B, _L, _H)


# speedup vs baseline: 1.0609x; 1.0609x over previous
import functools

import jax
import jax.numpy as jnp
from jax import lax
from jax.experimental import pallas as pl
from jax.experimental.pallas import tpu as pltpu
from jax.experimental.pallas import tpu_sc as plsc

_B, _L, _H = 1, 2048, 768
_E, _TOP_K, _FF = 16, 2, 512
_KCONV = 3
_EPS = 1e-05

_BLK1 = 256
_BLK2 = 256
_MAXS = _L * _TOP_K + _E * _BLK2
_NBLK = _MAXS // _BLK2
_NW = 32
_TPW = _L // _NW
_SPW = _MAXS // _NW

_mesh = plsc.VectorSubcoreMesh(core_axis_name="c", subcore_axis_name="s")
_sc_params = pltpu.CompilerParams(needs_layout_passes=False)


def _rms(x, w):
    return x * jax.lax.rsqrt(jnp.mean(x * x, axis=-1, keepdims=True) + _EPS) * w


def _stage1_body(x_ref, opw_ref, ffnw_ref, win_ref, wconvt_ref, wout_ref,
                 wr_ref, bias_ref, h_ref, t_ref, comb_ref, carry_ref,
                 winb_ref, woutb_ref):
    i = pl.program_id(0)

    @pl.when(i == 0)
    def _():
        carry_ref[...] = jnp.zeros_like(carry_ref)
        winb_ref[...] = win_ref[...].astype(jnp.bfloat16)
        woutb_ref[...] = wout_ref[...].astype(jnp.bfloat16)

    xb = x_ref[...]
    hn = _rms(xb, opw_ref[...])
    bcx = jnp.dot(hn.astype(jnp.bfloat16), winb_ref[...],
                  preferred_element_type=jnp.float32)
    b_ = bcx[:, :_H]
    c_ = bcx[:, _H:2 * _H]
    xc = bcx[:, 2 * _H:]
    Bx = b_ * xc

    carry = carry_ref[...]
    sh1 = jnp.concatenate([carry[7:8], Bx[:-1]], axis=0)
    sh2 = jnp.concatenate([carry[6:8], Bx[:-2]], axis=0)[:_BLK1]
    w0 = wconvt_ref[0:1, :]
    w1 = wconvt_ref[1:2, :]
    w2 = wconvt_ref[2:3, :]
    conv = sh2 * w0 + sh1 * w1 + Bx * w2
    carry_ref[...] = Bx[-8:]

    y = c_ * conv
    r = jnp.dot(y.astype(jnp.bfloat16), woutb_ref[...],
                preferred_element_type=jnp.float32)
    h = xb + r
    h_ref[...] = h

    t = _rms(h, ffnw_ref[...])
    t_ref[...] = t

    logits = jnp.dot(t, wr_ref[...], preferred_element_type=jnp.float32)
    m = jnp.max(logits, axis=-1, keepdims=True)
    p = jnp.exp(logits - m)
    g = p / jnp.sum(p, axis=-1, keepdims=True) + bias_ref[...]

    iota = jax.lax.broadcasted_iota(jnp.int32, g.shape, 1)
    m0 = jnp.max(g, axis=-1, keepdims=True)
    i0 = jnp.min(jnp.where(g == m0, iota, _E), axis=-1, keepdims=True)
    oh0 = iota == i0
    g1 = jnp.where(oh0, -jnp.inf, g)
    m1 = jnp.max(g1, axis=-1, keepdims=True)
    i1 = jnp.min(jnp.where(g1 == m1, iota, _E), axis=-1, keepdims=True)
    oh1 = iota == i1
    denom = m0 + m1 + 1e-20
    comb_ref[...] = (jnp.where(oh0, m0, 0.0) + jnp.where(oh1, m1, 0.0)) / denom


def _wid():
    return lax.axis_index("s") * 2 + lax.axis_index("c")


def _iota16():
    return lax.iota(jnp.int32, 16)


def _route_body(comb_hbm, svals_hbm, eids_hbm, cnts_hbm,
                comb_v, svals_v, eids_v, cnt_v, sem):
    w = _wid()
    tok0 = w * _TPW
    pltpu.sync_copy(comb_hbm.at[pl.ds(tok0, _TPW)], comb_v)
    iota = _iota16()
    mask2 = iota < _TOP_K
    cnt_v[...] = jnp.zeros((16,), jnp.int32)
    ones = jnp.ones((16,), jnp.int32)
    for j in range(_TPW):
        row = comb_v[j]
        sv, se = plsc.sort_key_val(row, iota, descending=True)
        tot = jnp.sum(jnp.where(mask2, sv, 0.0))
        svals_v[j] = sv / (tot + 1e-20)
        eids_v[j] = se
        plsc.addupdate_scatter(cnt_v, [se], ones, mask=mask2)
    pltpu.sync_copy(svals_v, svals_hbm.at[pl.ds(tok0, _TPW)])
    pltpu.sync_copy(eids_v, eids_hbm.at[pl.ds(tok0, _TPW)])
    pltpu.sync_copy(cnt_v, cnts_hbm.at[w])


def _slot_body(cnts_hbm, eids_hbm, t_hbm, slotmap_hbm, blkinfo_hbm, td_hbm,
               cnts_v, eids_v, cntrun_v, slotmap_v, s0pad_v, s1pad_v,
               slot0_v, slot1_v, trows_v, blk_v, nb_v, sem, sem2):
    w = _wid()
    tok0 = w * _TPW
    cp_t = pltpu.async_copy(t_hbm.at[pl.ds(tok0, _TPW)], trows_v, sem)
    pltpu.sync_copy(cnts_hbm, cnts_v)
    pltpu.sync_copy(eids_hbm.at[pl.ds(tok0, _TPW)], eids_v)
    iota = _iota16()
    mask2 = iota < _TOP_K
    ones = jnp.ones((16,), jnp.int32)
    zeros16 = jnp.zeros((16,), jnp.int32)

    gcnt = jnp.zeros((16,), jnp.int32)
    pref = jnp.zeros((16,), jnp.int32)
    for wp in range(_NW):
        rowc = cnts_v[wp]
        gcnt = gcnt + rowc
        pref = pref + jnp.where(wp < w, rowc, zeros16)
    pc = ((gcnt + (_BLK2 - 1)) >> 8) << 8
    base = plsc.cumsum(pc) - pc
    mybase = base + pref
    cntrun_v[...] = jnp.zeros((16,), jnp.int32)

    for j in range(_TPW):
        ev = eids_v[j]
        r = plsc.load_gather(cntrun_v, [ev], mask=mask2)
        b = jnp.take_along_axis(mybase, ev, axis=0)
        plsc.addupdate_scatter(cntrun_v, [ev], ones, mask=mask2)
        slotv = b + r
        slotmap_v[j] = slotv
        plsc.store_compressed(s0pad_v.at[pl.ds(j, 16)], slotv, mask=iota == 0)
        plsc.store_compressed(s1pad_v.at[pl.ds(j, 16)], slotv, mask=iota == 1)

    for g in range(_TPW // 16):
        slot0_v[pl.ds(16 * g, 16)] = s0pad_v[pl.ds(16 * g, 16)]
        slot1_v[pl.ds(16 * g, 16)] = s1pad_v[pl.ds(16 * g, 16)]

    pltpu.sync_copy(slotmap_v, slotmap_hbm.at[pl.ds(tok0, _TPW)])
    cp_t.wait()
    c0 = pltpu.async_copy(trows_v, td_hbm.at[slot0_v], sem)
    c1 = pltpu.async_copy(trows_v, td_hbm.at[slot1_v], sem2)

    @pl.when(w == 0)
    def _():
        bstart = base >> 8
        for c in range(_NBLK // 16):
            bvec = iota + 16 * c
            cntv = jnp.zeros((16,), jnp.int32)
            for e in range(_E):
                bse = jnp.take_along_axis(
                    bstart, jnp.full((16,), e, jnp.int32), axis=0)
                cntv = cntv + jnp.where(bvec >= bse, 1, 0).astype(jnp.int32)
            blk_v[pl.ds(16 * c, 16)] = jnp.clip(cntv - 1, 0, _E - 1)
        nb = jnp.sum(pc) >> 8
        nb_v[...] = jnp.full((16,), nb, jnp.int32)
        pltpu.sync_copy(blk_v, blkinfo_hbm.at[pl.ds(0, _NBLK)])
        pltpu.sync_copy(nb_v, blkinfo_hbm.at[pl.ds(64, 16)])

    c0.wait()
    c1.wait()


def _ffn_body(binfo_ref, td_ref, wg_ref, wu_ref, wd_ref, y_ref,
              wgc_ref, wuc_ref, wdc_ref):
    i = pl.program_id(0)

    @pl.when(i < binfo_ref[64])
    def _():
        prev = binfo_ref[jnp.maximum(i - 1, 0)]

        @pl.when((i == 0) | (binfo_ref[i] != prev))
        def _():
            wgc_ref[...] = wg_ref[0].astype(jnp.bfloat16)
            wuc_ref[...] = wu_ref[0].astype(jnp.bfloat16)
            wdc_ref[...] = wd_ref[0].astype(jnp.bfloat16)

        td = td_ref[...].astype(jnp.bfloat16)
        a = jnp.dot(td, wgc_ref[...], preferred_element_type=jnp.float32)
        u = jnp.dot(td, wuc_ref[...], preferred_element_type=jnp.float32)
        he = (a / (1.0 + jnp.exp(-a))) * u
        y_ref[...] = jnp.dot(he.astype(jnp.bfloat16), wdc_ref[...],
                             preferred_element_type=jnp.float32)


def _combine_body(slotmap_hbm, svals_hbm, y_hbm, h_hbm, out_hbm,
                  slm_v, sv_v, sl2_v, yr_v, h_v, out_v,
                  ssv, sy0, sy1, sh0, sh1, so0, so1):
    w = _wid()
    iota = _iota16()
    tokb = w * _TPW
    cps = pltpu.async_copy(svals_hbm.at[pl.ds(tokb, _TPW)], sv_v, ssv)
    pltpu.sync_copy(slotmap_hbm.at[pl.ds(tokb, _TPW)], slm_v)
    for c in range(4):
        for gg in range(2):
            g = 2 * c + gg
            rvec = (iota >> 1) + 8 * g
            sl2_v[c, pl.ds(16 * gg, 16)] = plsc.load_gather(
                slm_v, [rvec, iota & 1])

    sy = (sy0, sy1)
    sh = (sh0, sh1)
    so = (so0, so1)
    cpy = [None, None]
    cph = [None, None]
    cpo = [None, None]
    cpy[0] = pltpu.async_copy(y_hbm.at[sl2_v.at[0]], yr_v.at[0], sy[0])
    cph[0] = pltpu.async_copy(h_hbm.at[pl.ds(tokb, 16)], h_v.at[0], sh[0])
    cps.wait()
    for c in range(4):
        b = c & 1
        if c + 1 < 4:
            nb = (c + 1) & 1
            cpy[nb] = pltpu.async_copy(
                y_hbm.at[sl2_v.at[c + 1]], yr_v.at[nb], sy[nb])
            cph[nb] = pltpu.async_copy(
                h_hbm.at[pl.ds(tokb + 16 * (c + 1), 16)], h_v.at[nb], sh[nb])
        cpy[b].wait()
        cph[b].wait()
        if c >= 2:
            cpo[b].wait()

        def tok(j, carry, c=c, b=b):
            s0 = plsc.load_gather(sv_v, [jnp.full((16,), c * 16 + j, jnp.int32),
                                         jnp.zeros((16,), jnp.int32)])
            s1 = plsc.load_gather(sv_v, [jnp.full((16,), c * 16 + j, jnp.int32),
                                         jnp.ones((16,), jnp.int32)])
            for k in range(_H // 16):
                sl = pl.ds(16 * k, 16)
                out_v[b, j, sl] = (h_v[b, j, sl] + s0 * yr_v[b, 2 * j, sl]
                                   + s1 * yr_v[b, 2 * j + 1, sl])
            return carry

        lax.fori_loop(0, 16, tok, 0)
        cpo[b] = pltpu.async_copy(
            out_v.at[b], out_hbm.at[pl.ds(tokb + 16 * c, 16)], so[b])
    cpo[0].wait()
    cpo[1].wait()


def kernel(x, op_norm_w, ffn_norm_w, w_in, w_conv, w_out, w_router,
           expert_bias, w_g, w_u, w_d):
    x2 = x.reshape(_L, _H)
    wconvt = jnp.zeros((8, _H), jnp.float32).at[:_KCONV].set(w_conv.T)

    nblk1 = _L // _BLK1
    h, t, comb = pl.pallas_call(
        _stage1_body,
        grid=(nblk1,),
        in_specs=[
            pl.BlockSpec((_BLK1, _H), lambda i: (i, 0)),
            pl.BlockSpec((1, _H), lambda i: (0, 0)),
            pl.BlockSpec((1, _H), lambda i: (0, 0)),
            pl.BlockSpec((_H, 3 * _H), lambda i: (0, 0)),
            pl.BlockSpec((8, _H), lambda i: (0, 0)),
            pl.BlockSpec((_H, _H), lambda i: (0, 0)),
            pl.BlockSpec((_H, _E), lambda i: (0, 0)),
            pl.BlockSpec((1, _E), lambda i: (0, 0)),
        ],
        out_specs=[
            pl.BlockSpec((_BLK1, _H), lambda i: (i, 0)),
            pl.BlockSpec((_BLK1, _H), lambda i: (i, 0)),
            pl.BlockSpec((_BLK1, _E), lambda i: (i, 0)),
        ],
        out_shape=[
            jax.ShapeDtypeStruct((_L, _H), jnp.float32),
            jax.ShapeDtypeStruct((_L, _H), jnp.float32),
            jax.ShapeDtypeStruct((_L, _E), jnp.float32),
        ],
        scratch_shapes=[
            pltpu.VMEM((8, _H), jnp.float32),
            pltpu.VMEM((_H, 3 * _H), jnp.bfloat16),
            pltpu.VMEM((_H, _H), jnp.bfloat16),
        ],
    )(
        x2,
        op_norm_w.reshape(1, _H),
        ffn_norm_w.reshape(1, _H),
        w_in,
        wconvt,
        w_out,
        w_router,
        expert_bias.reshape(1, _E),
    )

    svals, eids, cnts = pl.kernel(
        _route_body,
        mesh=_mesh,
        compiler_params=_sc_params,
        out_type=[
            jax.ShapeDtypeStruct((_L, 16), jnp.float32),
            jax.ShapeDtypeStruct((_L, 16), jnp.int32),
            jax.ShapeDtypeStruct((_NW, 16), jnp.int32),
        ],
        scratch_types=[
            pltpu.VMEM((_TPW, 16), jnp.float32),
            pltpu.VMEM((_TPW, 16), jnp.float32),
            pltpu.VMEM((_TPW, 16), jnp.int32),
            pltpu.VMEM((16,), jnp.int32),
            pltpu.SemaphoreType.DMA,
        ],
    )(comb)

    slotmap, blkinfo, td = pl.kernel(
        _slot_body,
        mesh=_mesh,
        compiler_params=_sc_params,
        out_type=[
            jax.ShapeDtypeStruct((_L, 16), jnp.int32),
            jax.ShapeDtypeStruct((128,), jnp.int32),
            jax.ShapeDtypeStruct((_MAXS, _H), jnp.float32),
        ],
        scratch_types=[
            pltpu.VMEM((_NW, 16), jnp.int32),
            pltpu.VMEM((_TPW, 16), jnp.int32),
            pltpu.VMEM((16,), jnp.int32),
            pltpu.VMEM((_TPW, 16), jnp.int32),
            pltpu.VMEM((_TPW + 16, ), jnp.int32),
            pltpu.VMEM((_TPW + 16, ), jnp.int32),
            pltpu.VMEM((_TPW,), jnp.int32),
            pltpu.VMEM((_TPW,), jnp.int32),
            pltpu.VMEM((_TPW, _H), jnp.float32),
            pltpu.VMEM((_NBLK,), jnp.int32),
            pltpu.VMEM((16,), jnp.int32),
            pltpu.SemaphoreType.DMA,
            pltpu.SemaphoreType.DMA,
        ],
    )(cnts, eids, t)

    y = pl.pallas_call(
        _ffn_body,
        grid_spec=pltpu.PrefetchScalarGridSpec(
            num_scalar_prefetch=1,
            grid=(_NBLK,),
            in_specs=[
                pl.BlockSpec((_BLK2, _H), lambda i, binfo: (i, 0)),
                pl.BlockSpec((1, _H, _FF), lambda i, binfo: (binfo[i], 0, 0)),
                pl.BlockSpec((1, _H, _FF), lambda i, binfo: (binfo[i], 0, 0)),
                pl.BlockSpec((1, _FF, _H), lambda i, binfo: (binfo[i], 0, 0)),
            ],
            out_specs=pl.BlockSpec((_BLK2, _H), lambda i, binfo: (i, 0)),
            scratch_shapes=[
                pltpu.VMEM((_H, _FF), jnp.bfloat16),
                pltpu.VMEM((_H, _FF), jnp.bfloat16),
                pltpu.VMEM((_FF, _H), jnp.bfloat16),
            ],
        ),
        out_shape=jax.ShapeDtypeStruct((_MAXS, _H), jnp.float32),
    )(blkinfo, td, w_g, w_u, w_d)

    out = pl.kernel(
        _combine_body,
        mesh=_mesh,
        compiler_params=_sc_params,
        out_type=jax.ShapeDtypeStruct((_L, _H), jnp.float32),
        scratch_types=[
            pltpu.VMEM((_TPW, 16), jnp.int32),
            pltpu.VMEM((_TPW, 16), jnp.float32),
            pltpu.VMEM((4, 32), jnp.int32),
            pltpu.VMEM((2, 32, _H), jnp.float32),
            pltpu.VMEM((2, 16, _H), jnp.float32),
            pltpu.VMEM((2, 16, _H), jnp.float32),
            pltpu.SemaphoreType.DMA,
            pltpu.SemaphoreType.DMA,
            pltpu.SemaphoreType.DMA,
            pltpu.SemaphoreType.DMA,
            pltpu.SemaphoreType.DMA,
            pltpu.SemaphoreType.DMA,
            pltpu.SemaphoreType.DMA,
        ],
    )(slotmap, svals, y, h)

    return out.reshape(_B, _L, _H)


# speedup vs baseline: 1.0629x; 1.0019x over previous
import jax
import jax.numpy as jnp
from jax import lax
from jax.experimental import pallas as pl
from jax.experimental.pallas import tpu as pltpu
from jax.experimental.pallas import tpu_sc as plsc

_B, _L, _H = 1, 2048, 768
_E, _TOP_K, _FF = 16, 2, 512
_KCONV = 3
_EPS = 1e-05

_BLK1 = 256
_BLK2 = 256
_MAXS = _L * _TOP_K + _E * _BLK2
_NBLK = _MAXS // _BLK2
_NW = 32
_TPW = _L // _NW
_SPW = _MAXS // _NW

_mesh = plsc.VectorSubcoreMesh(core_axis_name="c", subcore_axis_name="s")
_sc_params = pltpu.CompilerParams(needs_layout_passes=False)


def _rms(x, w):
    return x * jax.lax.rsqrt(jnp.mean(x * x, axis=-1, keepdims=True) + _EPS) * w


def _stage1_body(x_ref, opw_ref, ffnw_ref, win_ref, wconvt_ref, wout_ref,
                 wr_ref, bias_ref, h_ref, t_ref, comb_ref, carry_ref,
                 winb_ref, woutb_ref):
    i = pl.program_id(0)

    @pl.when(i == 0)
    def _():
        carry_ref[...] = jnp.zeros_like(carry_ref)
        winb_ref[...] = win_ref[...].astype(jnp.bfloat16)
        woutb_ref[...] = wout_ref[...].astype(jnp.bfloat16)

    xb = x_ref[...]
    hn = _rms(xb, opw_ref[...])
    bcx = jnp.dot(hn.astype(jnp.bfloat16), winb_ref[...],
                  preferred_element_type=jnp.float32)
    b_ = bcx[:, :_H]
    c_ = bcx[:, _H:2 * _H]
    xc = bcx[:, 2 * _H:]
    Bx = b_ * xc

    carry = carry_ref[...]
    sh1 = jnp.concatenate([carry[7:8], Bx[:-1]], axis=0)
    sh2 = jnp.concatenate([carry[6:8], Bx[:-2]], axis=0)[:_BLK1]
    w0 = wconvt_ref[0:1, :]
    w1 = wconvt_ref[1:2, :]
    w2 = wconvt_ref[2:3, :]
    conv = sh2 * w0 + sh1 * w1 + Bx * w2
    carry_ref[...] = Bx[-8:]

    y = c_ * conv
    r = jnp.dot(y.astype(jnp.bfloat16), woutb_ref[...],
                preferred_element_type=jnp.float32)
    h = xb + r
    h_ref[...] = h

    t = _rms(h, ffnw_ref[...])
    t_ref[...] = t

    logits = jnp.dot(t, wr_ref[...], preferred_element_type=jnp.float32)
    m = jnp.max(logits, axis=-1, keepdims=True)
    p = jnp.exp(logits - m)
    g = p / jnp.sum(p, axis=-1, keepdims=True) + bias_ref[...]

    iota = jax.lax.broadcasted_iota(jnp.int32, g.shape, 1)
    m0 = jnp.max(g, axis=-1, keepdims=True)
    i0 = jnp.min(jnp.where(g == m0, iota, _E), axis=-1, keepdims=True)
    oh0 = iota == i0
    g1 = jnp.where(oh0, -jnp.inf, g)
    m1 = jnp.max(g1, axis=-1, keepdims=True)
    i1 = jnp.min(jnp.where(g1 == m1, iota, _E), axis=-1, keepdims=True)
    oh1 = iota == i1
    denom = m0 + m1 + 1e-20
    comb_ref[...] = (jnp.where(oh0, m0, 0.0) + jnp.where(oh1, m1, 0.0)) / denom


def _wid():
    return lax.axis_index("s") * 2 + lax.axis_index("c")


def _iota16():
    return lax.iota(jnp.int32, 16)


def _route_body(comb_hbm, svals_hbm, eids_hbm, cnts_hbm,
                comb_v, svals_v, eids_v, cnt_v, sem):
    w = _wid()
    tok0 = w * _TPW
    pltpu.sync_copy(comb_hbm.at[pl.ds(tok0, _TPW)], comb_v)
    iota = _iota16()
    mask2 = iota < _TOP_K
    cnt_v[...] = jnp.zeros((16,), jnp.int32)
    ones = jnp.ones((16,), jnp.int32)
    for j in range(_TPW):
        row = comb_v[j]
        sv, se = plsc.sort_key_val(row, iota, descending=True)
        tot = jnp.sum(jnp.where(mask2, sv, 0.0))
        svals_v[j] = sv / (tot + 1e-20)
        eids_v[j] = se
        plsc.addupdate_scatter(cnt_v, [se], ones, mask=mask2)
    pltpu.sync_copy(svals_v, svals_hbm.at[pl.ds(tok0, _TPW)])
    pltpu.sync_copy(eids_v, eids_hbm.at[pl.ds(tok0, _TPW)])
    pltpu.sync_copy(cnt_v, cnts_hbm.at[w])


def _slot_body(cnts_hbm, eids_hbm, t_hbm, slotmap_hbm, blkinfo_hbm, td_hbm,
               cnts_v, eids_v, cntrun_v, slotmap_v, s0pad_v, s1pad_v,
               slot0_v, slot1_v, trows_v, blk_v, nb_v, sem, sem2):
    w = _wid()
    tok0 = w * _TPW
    cp_t = pltpu.async_copy(t_hbm.at[pl.ds(tok0, _TPW)], trows_v, sem)
    pltpu.sync_copy(cnts_hbm, cnts_v)
    pltpu.sync_copy(eids_hbm.at[pl.ds(tok0, _TPW)], eids_v)
    iota = _iota16()
    mask2 = iota < _TOP_K
    ones = jnp.ones((16,), jnp.int32)
    zeros16 = jnp.zeros((16,), jnp.int32)

    gcnt = jnp.zeros((16,), jnp.int32)
    pref = jnp.zeros((16,), jnp.int32)
    for wp in range(_NW):
        rowc = cnts_v[wp]
        gcnt = gcnt + rowc
        pref = pref + jnp.where(wp < w, rowc, zeros16)
    pc = ((gcnt + (_BLK2 - 1)) >> 8) << 8
    base = plsc.cumsum(pc) - pc
    mybase = base + pref
    cntrun_v[...] = jnp.zeros((16,), jnp.int32)

    for j in range(_TPW):
        ev = eids_v[j]
        r = plsc.load_gather(cntrun_v, [ev], mask=mask2)
        b = jnp.take_along_axis(mybase, ev, axis=0)
        plsc.addupdate_scatter(cntrun_v, [ev], ones, mask=mask2)
        slotv = b + r
        slotmap_v[j] = slotv
        plsc.store_compressed(s0pad_v.at[pl.ds(j, 16)], slotv, mask=iota == 0)
        plsc.store_compressed(s1pad_v.at[pl.ds(j, 16)], slotv, mask=iota == 1)

    for g in range(_TPW // 16):
        slot0_v[pl.ds(16 * g, 16)] = s0pad_v[pl.ds(16 * g, 16)]
        slot1_v[pl.ds(16 * g, 16)] = s1pad_v[pl.ds(16 * g, 16)]

    pltpu.sync_copy(slotmap_v, slotmap_hbm.at[pl.ds(tok0, _TPW)])
    cp_t.wait()
    c0 = pltpu.async_copy(trows_v, td_hbm.at[slot0_v], sem)
    c1 = pltpu.async_copy(trows_v, td_hbm.at[slot1_v], sem2)

    @pl.when(w == 0)
    def _():
        bstart = base >> 8
        for c in range(_NBLK // 16):
            bvec = iota + 16 * c
            cntv = jnp.zeros((16,), jnp.int32)
            for e in range(_E):
                bse = jnp.take_along_axis(
                    bstart, jnp.full((16,), e, jnp.int32), axis=0)
                cntv = cntv + jnp.where(bvec >= bse, 1, 0).astype(jnp.int32)
            blk_v[pl.ds(16 * c, 16)] = jnp.clip(cntv - 1, 0, _E - 1)
        nb = jnp.sum(pc) >> 8
        nb_v[...] = jnp.full((16,), nb, jnp.int32)
        pltpu.sync_copy(blk_v, blkinfo_hbm.at[pl.ds(0, _NBLK)])
        pltpu.sync_copy(nb_v, blkinfo_hbm.at[pl.ds(64, 16)])

    c0.wait()
    c1.wait()


def _ffn_body(binfo_ref, td_ref, wg_ref, wu_ref, wd_ref, y_ref,
              wgc_ref, wuc_ref, wdc_ref):
    i = pl.program_id(0)

    @pl.when(i < binfo_ref[64])
    def _():
        prev = binfo_ref[jnp.maximum(i - 1, 0)]

        @pl.when((i == 0) | (binfo_ref[i] != prev))
        def _():
            wgc_ref[...] = wg_ref[0].astype(jnp.bfloat16)
            wuc_ref[...] = wu_ref[0].astype(jnp.bfloat16)
            wdc_ref[...] = wd_ref[0].astype(jnp.bfloat16)

        td = td_ref[...].astype(jnp.bfloat16)
        a = jnp.dot(td, wgc_ref[...], preferred_element_type=jnp.float32)
        u = jnp.dot(td, wuc_ref[...], preferred_element_type=jnp.float32)
        he = (a / (1.0 + jnp.exp(-a))) * u
        y_ref[...] = jnp.dot(he.astype(jnp.bfloat16), wdc_ref[...],
                             preferred_element_type=jnp.float32)


def _combine_body(slotmap_hbm, svals_hbm, y_hbm, h_hbm, out_hbm,
                  slm_v, sv_v, sl2_v, yr_v, h_v, out_v,
                  ssv, sy0, sy1, sh0, sh1, so0, so1):
    w = _wid()
    iota = _iota16()
    tokb = w * _TPW
    cps = pltpu.async_copy(svals_hbm.at[pl.ds(tokb, _TPW)], sv_v, ssv)
    pltpu.sync_copy(slotmap_hbm.at[pl.ds(tokb, _TPW)], slm_v)
    for c in range(4):
        for gg in range(2):
            g = 2 * c + gg
            rvec = (iota >> 1) + 8 * g
            sl2_v[c, pl.ds(16 * gg, 16)] = plsc.load_gather(
                slm_v, [rvec, iota & 1])

    sy = (sy0, sy1)
    sh = (sh0, sh1)
    so = (so0, so1)
    cpy = [None, None]
    cph = [None, None]
    cpo = [None, None]
    cpy[0] = pltpu.async_copy(y_hbm.at[sl2_v.at[0]], yr_v.at[0], sy[0])
    cph[0] = pltpu.async_copy(h_hbm.at[pl.ds(tokb, 16)], h_v.at[0], sh[0])
    cps.wait()
    for c in range(4):
        b = c & 1
        if c + 1 < 4:
            nb = (c + 1) & 1
            cpy[nb] = pltpu.async_copy(
                y_hbm.at[sl2_v.at[c + 1]], yr_v.at[nb], sy[nb])
            cph[nb] = pltpu.async_copy(
                h_hbm.at[pl.ds(tokb + 16 * (c + 1), 16)], h_v.at[nb], sh[nb])
        cpy[b].wait()
        cph[b].wait()
        if c >= 2:
            cpo[b].wait()

        def tok(j, carry, c=c, b=b):
            s0 = plsc.load_gather(sv_v, [jnp.full((16,), c * 16 + j, jnp.int32),
                                         jnp.zeros((16,), jnp.int32)])
            s1 = plsc.load_gather(sv_v, [jnp.full((16,), c * 16 + j, jnp.int32),
                                         jnp.ones((16,), jnp.int32)])
            for k in range(_H // 16):
                sl = pl.ds(16 * k, 16)
                out_v[b, j, sl] = (h_v[b, j, sl] + s0 * yr_v[b, 2 * j, sl]
                                   + s1 * yr_v[b, 2 * j + 1, sl])
            return carry

        lax.fori_loop(0, 16, tok, 0)
        cpo[b] = pltpu.async_copy(
            out_v.at[b], out_hbm.at[pl.ds(tokb + 16 * c, 16)], so[b])
    cpo[0].wait()
    cpo[1].wait()


def kernel(x, op_norm_w, ffn_norm_w, w_in, w_conv, w_out, w_router,
           expert_bias, w_g, w_u, w_d):
    x2 = x.reshape(_L, _H)
    wconvt = jnp.zeros((8, _H), jnp.float32).at[:_KCONV].set(w_conv.T)

    nblk1 = _L // _BLK1
    h, t, comb = pl.pallas_call(
        _stage1_body,
        grid=(nblk1,),
        in_specs=[
            pl.BlockSpec((_BLK1, _H), lambda i: (i, 0)),
            pl.BlockSpec((1, _H), lambda i: (0, 0)),
            pl.BlockSpec((1, _H), lambda i: (0, 0)),
            pl.BlockSpec((_H, 3 * _H), lambda i: (0, 0)),
            pl.BlockSpec((8, _H), lambda i: (0, 0)),
            pl.BlockSpec((_H, _H), lambda i: (0, 0)),
            pl.BlockSpec((_H, _E), lambda i: (0, 0)),
            pl.BlockSpec((1, _E), lambda i: (0, 0)),
        ],
        out_specs=[
            pl.BlockSpec((_BLK1, _H), lambda i: (i, 0)),
            pl.BlockSpec((_BLK1, _H), lambda i: (i, 0)),
            pl.BlockSpec((_BLK1, _E), lambda i: (i, 0)),
        ],
        out_shape=[
            jax.ShapeDtypeStruct((_L, _H), jnp.float32),
            jax.ShapeDtypeStruct((_L, _H), jnp.float32),
            jax.ShapeDtypeStruct((_L, _E), jnp.float32),
        ],
        scratch_shapes=[
            pltpu.VMEM((8, _H), jnp.float32),
            pltpu.VMEM((_H, 3 * _H), jnp.bfloat16),
            pltpu.VMEM((_H, _H), jnp.bfloat16),
        ],
    )(
        x2,
        op_norm_w.reshape(1, _H),
        ffn_norm_w.reshape(1, _H),
        w_in,
        wconvt,
        w_out,
        w_router,
        expert_bias.reshape(1, _E),
    )

    svals, eids, cnts = pl.kernel(
        _route_body,
        mesh=_mesh,
        compiler_params=_sc_params,
        out_type=[
            jax.ShapeDtypeStruct((_L, 16), jnp.float32),
            jax.ShapeDtypeStruct((_L, 16), jnp.int32),
            jax.ShapeDtypeStruct((_NW, 16), jnp.int32),
        ],
        scratch_types=[
            pltpu.VMEM((_TPW, 16), jnp.float32),
            pltpu.VMEM((_TPW, 16), jnp.float32),
            pltpu.VMEM((_TPW, 16), jnp.int32),
            pltpu.VMEM((16,), jnp.int32),
            pltpu.SemaphoreType.DMA,
        ],
    )(comb)

    slotmap, blkinfo, td = pl.kernel(
        _slot_body,
        mesh=_mesh,
        compiler_params=_sc_params,
        out_type=[
            jax.ShapeDtypeStruct((_L, 16), jnp.int32),
            jax.ShapeDtypeStruct((128,), jnp.int32),
            jax.ShapeDtypeStruct((_MAXS, _H), jnp.float32),
        ],
        scratch_types=[
            pltpu.VMEM((_NW, 16), jnp.int32),
            pltpu.VMEM((_TPW, 16), jnp.int32),
            pltpu.VMEM((16,), jnp.int32),
            pltpu.VMEM((_TPW, 16), jnp.int32),
            pltpu.VMEM((_TPW + 16, ), jnp.int32),
            pltpu.VMEM((_TPW + 16, ), jnp.int32),
            pltpu.VMEM((_TPW,), jnp.int32),
            pltpu.VMEM((_TPW,), jnp.int32),
            pltpu.VMEM((_TPW, _H), jnp.float32),
            pltpu.VMEM((_NBLK,), jnp.int32),
            pltpu.VMEM((16,), jnp.int32),
            pltpu.SemaphoreType.DMA,
            pltpu.SemaphoreType.DMA,
        ],
    )(cnts, eids, t)

    y = pl.pallas_call(
        _ffn_body,
        grid_spec=pltpu.PrefetchScalarGridSpec(
            num_scalar_prefetch=1,
            grid=(_NBLK,),
            in_specs=[
                pl.BlockSpec((_BLK2, _H), lambda i, binfo: (i, 0)),
                pl.BlockSpec((1, _H, _FF), lambda i, binfo: (binfo[i], 0, 0)),
                pl.BlockSpec((1, _H, _FF), lambda i, binfo: (binfo[i], 0, 0)),
                pl.BlockSpec((1, _FF, _H), lambda i, binfo: (binfo[i], 0, 0)),
            ],
            out_specs=pl.BlockSpec((_BLK2, _H), lambda i, binfo: (i, 0)),
            scratch_shapes=[
                pltpu.VMEM((_H, _FF), jnp.bfloat16),
                pltpu.VMEM((_H, _FF), jnp.bfloat16),
                pltpu.VMEM((_FF, _H), jnp.bfloat16),
            ],
        ),
        out_shape=jax.ShapeDtypeStruct((_MAXS, _H), jnp.float32),
    )(blkinfo, td, w_g, w_u, w_d)

    out = pl.kernel(
        _combine_body,
        mesh=_mesh,
        compiler_params=_sc_params,
        out_type=jax.ShapeDtypeStruct((_L, _H), jnp.float32),
        scratch_types=[
            pltpu.VMEM((_TPW, 16), jnp.int32),
            pltpu.VMEM((_TPW, 16), jnp.float32),
            pltpu.VMEM((4, 32), jnp.int32),
            pltpu.VMEM((2, 32, _H), jnp.float32),
            pltpu.VMEM((2, 16, _H), jnp.float32),
            pltpu.VMEM((2, 16, _H), jnp.float32),
            pltpu.SemaphoreType.DMA,
            pltpu.SemaphoreType.DMA,
            pltpu.SemaphoreType.DMA,
            pltpu.SemaphoreType.DMA,
            pltpu.SemaphoreType.DMA,
            pltpu.SemaphoreType.DMA,
            pltpu.SemaphoreType.DMA,
        ],
    )(slotmap, svals, y, h)

    return out.reshape(_B, _L, _H)
